# Initial kernel scaffold; baseline (speedup 1.0000x reference)
import jax
import jax.numpy as jnp
from jax.experimental import pallas as pl


def kernel(x, edge_index, edge_attr, batch, r_target, params):
    raise NotImplementedError("write your pallas kernel here")


# speedup vs baseline: 8.1948x; 8.1948x over previous
import functools

import jax
import jax.numpy as jnp
import numpy as np
from jax import lax
from jax.experimental import pallas as pl
from jax.experimental.pallas import tpu as pltpu
from jax.experimental.pallas import tpu_sc as plsc

N = 10000
E = 320000
D = 128
H = 4
C = 32
HC = H * C
G = 256
T = 4
DE = 4

ACC_W = 160
A_W = 144
B_W = 256
NP = 10240
NW = 32
BE = 128
SB = 32
EPAD = 323584
EW = EPAD // NW
NB = EW // BE
CH = SB // 16

_f32 = jnp.float32
_i32 = jnp.int32


def _edge_body(a_hbm, b_hbm, src_hbm, dst_hbm, ea_hbm, zrs_hbm, out_hbm,
               srcv_sb, dstv_sb, eav, arows, brows, orows, bounce, acc,
               sem0, sem1):
    cid = lax.axis_index("c")
    sid = lax.axis_index("s")
    rows_per_tile = NP // 16
    row0 = sid * rows_per_tile
    zv = jnp.zeros((16,), _f32)

    def zrow(r, _):
        def zcol(wd, _2):
            bounce[r, pl.ds(wd * 16, 16)] = zv
            return 0
        return lax.fori_loop(0, ACC_W // 16, zcol, 0)
    lax.fori_loop(0, SB, zrow, 0)

    def zcopy(i, _):
        pltpu.sync_copy(bounce, acc.at[pl.ds(row0 + i * SB, SB)])
        return 0
    lax.fori_loop(0, rows_per_tile // SB, zcopy, 0)
    plsc.subcore_barrier()

    w = sid * 2 + cid
    base_w = w * EW
    lane = lax.iota(_i32, 16)

    def block_body(b, _):
        base = base_w + b * BE
        pltpu.sync_copy(ea_hbm.at[:, pl.ds(base, BE)], eav)

        def sub_body(sb, _s0):
            sb0 = sb * SB
            pltpu.sync_copy(src_hbm.at[pl.ds(base + sb0, SB)], srcv_sb)
            pltpu.sync_copy(dst_hbm.at[pl.ds(base + sb0, SB)], dstv_sb)
            d0 = pltpu.async_copy(a_hbm.at[dstv_sb], arows, sem0)
            d1 = pltpu.async_copy(b_hbm.at[srcv_sb], brows, sem1)
            d0.wait()
            d1.wait()

            for j in range(CH):
                ridx = j * 16 + lane
                eac = [eav[d, pl.ds(sb0 + j * 16, 16)] for d in range(DE)]
                for h in range(H):
                    parts = [jnp.zeros((16,), _f32) for _ in range(4)]
                    for c2 in range(C):
                        col = jnp.full((16,), h * C + c2, _i32)
                        qc = plsc.load_gather(arows, [ridx, col])
                        kc = plsc.load_gather(brows, [ridx, col])
                        parts[c2 % 4] = parts[c2 % 4] + qc * kc
                    ah = (parts[0] + parts[1]) + (parts[2] + parts[3])
                    for d in range(DE):
                        qp = plsc.load_gather(
                            arows,
                            [ridx, jnp.full((16,), HC + h * DE + d, _i32)])
                        ah = ah + qp * eac[d]
                    ex_h = jnp.exp(ah)

                    for c2 in range(C):
                        vcol = jnp.full((16,), HC + h * C + c2, _i32)
                        ocol = jnp.full((16,), h * C + c2, _i32)
                        vc = plsc.load_gather(brows, [ridx, vcol])
                        plsc.store_scatter(orows, [ridx, ocol], vc * ex_h)
                    plsc.store_scatter(
                        orows, [ridx, jnp.full((16,), HC + h, _i32)], ex_h)
                    for d in range(DE):
                        plsc.store_scatter(
                            orows,
                            [ridx, jnp.full((16,), HC + 4 + h * DE + d, _i32)],
                            ex_h * eac[d])
            pltpu.sync_copy(orows, acc.at[dstv_sb], add=True)
            return 0

        lax.fori_loop(0, BE // SB, sub_body, 0)
        return 0

    lax.fori_loop(0, NB, block_body, 0)
    plsc.subcore_barrier()

    def ocopy(i, _):
        pltpu.sync_copy(acc.at[pl.ds(row0 + i * SB, SB)], bounce)
        pltpu.sync_copy(bounce, out_hbm.at[cid, pl.ds(row0 + i * SB, SB)])
        return 0
    lax.fori_loop(0, rows_per_tile // SB, ocopy, 0)


@functools.cache
def _edge_pass_fn():
    return pl.kernel(
        _edge_body,
        compiler_params=pltpu.CompilerParams(use_tc_tiling_on_sc=False,
                                             needs_layout_passes=False),
        out_type=jax.ShapeDtypeStruct((2, NP, ACC_W), _f32),
        mesh=plsc.VectorSubcoreMesh(core_axis_name="c", subcore_axis_name="s",
                                    num_cores=2, num_subcores=16),
        scratch_types=[
            pltpu.VMEM((SB,), _i32),
            pltpu.VMEM((SB,), _i32),
            pltpu.VMEM((DE, BE), _f32),
            pltpu.VMEM((SB, A_W), _f32),
            pltpu.VMEM((SB, B_W), _f32),
            pltpu.VMEM((SB, ACC_W), _f32),
            pltpu.VMEM((SB, ACC_W), _f32),
            pltpu.VMEM_SHARED((NP, ACC_W), _f32),
            pltpu.SemaphoreType.DMA,
            pltpu.SemaphoreType.DMA,
        ],
    )


def _edge_pass(a, bb, src, dst, ea_t, zrs):
    return _edge_pass_fn()(a, bb, src, dst, ea_t, zrs)


_BLK = 1000
_NBLK = N // _BLK


def _pre_body(x_ref, w_ref, b_ref, a_ref, bb_ref, s_ref):
    z = jnp.dot(x_ref[...], w_ref[...], preferred_element_type=_f32) + b_ref[...]
    a_ref[...] = z[:, 0:A_W]
    bb_ref[...] = z[:, A_W:A_W + B_W]
    s_ref[...] = z[:, A_W + B_W:]


def _pre_call(h, wcat, bcat):
    return pl.pallas_call(
        _pre_body,
        grid=(_NBLK,),
        in_specs=[
            pl.BlockSpec((_BLK, HC), lambda i: (i, 0)),
            pl.BlockSpec((HC, 528), lambda i: (0, 0)),
            pl.BlockSpec((1, 528), lambda i: (0, 0)),
        ],
        out_specs=[
            pl.BlockSpec((_BLK, A_W), lambda i: (i, 0)),
            pl.BlockSpec((_BLK, B_W), lambda i: (i, 0)),
            pl.BlockSpec((_BLK, HC), lambda i: (i, 0)),
        ],
        out_shape=[
            jax.ShapeDtypeStruct((NP, A_W), _f32),
            jax.ShapeDtypeStruct((NP, B_W), _f32),
            jax.ShapeDtypeStruct((N, HC), _f32),
        ],
    )(h, wcat, bcat)


def _node_out(acc_ref, s_ref, m_ref, r_ref):
    accs = acc_ref[0] + acc_ref[1]
    num = accs[:, 0:HC] + jnp.dot(accs[:, HC + 4:HC + 20], m_ref[...],
                                  preferred_element_type=_f32)
    den = jnp.dot(accs[:, HC:HC + 8], r_ref[...],
                  preferred_element_type=_f32) + 1e-16
    return num / den + s_ref[...]


def _postpre_body(acc_ref, s_ref, m_ref, r_ref, w_ref, b_ref,
                  a_ref, bb_ref, s2_ref):
    h = jnp.maximum(_node_out(acc_ref, s_ref, m_ref, r_ref), 0.0)
    z = jnp.dot(h, w_ref[...], preferred_element_type=_f32) + b_ref[...]
    a_ref[...] = z[:, 0:A_W]
    bb_ref[...] = z[:, A_W:A_W + B_W]
    s2_ref[...] = z[:, A_W + B_W:]


def _postpre_call(acc, s, m, r8, wcat, bcat):
    return pl.pallas_call(
        _postpre_body,
        grid=(_NBLK,),
        in_specs=[
            pl.BlockSpec((2, _BLK, ACC_W), lambda i: (0, i, 0)),
            pl.BlockSpec((_BLK, HC), lambda i: (i, 0)),
            pl.BlockSpec((16, HC), lambda i: (0, 0)),
            pl.BlockSpec((8, HC), lambda i: (0, 0)),
            pl.BlockSpec((HC, 528), lambda i: (0, 0)),
            pl.BlockSpec((1, 528), lambda i: (0, 0)),
        ],
        out_specs=[
            pl.BlockSpec((_BLK, A_W), lambda i: (i, 0)),
            pl.BlockSpec((_BLK, B_W), lambda i: (i, 0)),
            pl.BlockSpec((_BLK, HC), lambda i: (i, 0)),
        ],
        out_shape=[
            jax.ShapeDtypeStruct((NP, A_W), _f32),
            jax.ShapeDtypeStruct((NP, B_W), _f32),
            jax.ShapeDtypeStruct((N, HC), _f32),
        ],
    )(acc, s, m, r8, wcat, bcat)


def _final_body(acc_ref, s_ref, m_ref, r_ref, batch_ref, rt_ref,
                hw_ref, hb_ref, out_ref, pooled, cnt):
    i = pl.program_id(0)

    @pl.when(i == 0)
    def _init():
        pooled[...] = jnp.zeros_like(pooled)
        cnt[...] = jnp.zeros_like(cnt)

    h3 = _node_out(acc_ref, s_ref, m_ref, r_ref)
    b = batch_ref[0, 0, :]
    oh = (jnp.reshape(b, (_BLK, 1)) ==
          lax.broadcasted_iota(_i32, (_BLK, G), 1)).astype(_f32)
    dn = (((0,), (0,)), ((), ()))
    pooled[...] += lax.dot_general(oh, h3, dn, preferred_element_type=_f32)
    cnt[...] += lax.dot_general(oh, jnp.ones((_BLK, 8), _f32), dn,
                                preferred_element_type=_f32)

    @pl.when(i == _NBLK - 1)
    def _head():
        pm = pooled[...] / jnp.maximum(cnt[:, 0:1], 1.0)
        rt = rt_ref[:, 0:1]
        ohr = (rt == lax.broadcasted_iota(_i32, (G, 8), 1)).astype(_f32)
        whsel = jnp.dot(ohr, hw_ref[...], preferred_element_type=_f32)
        bhsel = jnp.dot(ohr, hb_ref[...], preferred_element_type=_f32)
        out_ref[...] = (jnp.sum(pm * whsel, axis=1, keepdims=True)
                        + bhsel[:, 0:1])


def _final_call(acc, s, m, r8, batch3d, rt2d, hw8, hb8):
    return pl.pallas_call(
        _final_body,
        grid=(_NBLK,),
        in_specs=[
            pl.BlockSpec((2, _BLK, ACC_W), lambda i: (0, i, 0)),
            pl.BlockSpec((_BLK, HC), lambda i: (i, 0)),
            pl.BlockSpec((16, HC), lambda i: (0, 0)),
            pl.BlockSpec((8, HC), lambda i: (0, 0)),
            pl.BlockSpec((1, 1, _BLK), lambda i: (i, 0, 0)),
            pl.BlockSpec((G, 8), lambda i: (0, 0)),
            pl.BlockSpec((8, HC), lambda i: (0, 0)),
            pl.BlockSpec((8, 8), lambda i: (0, 0)),
        ],
        out_specs=pl.BlockSpec((G, 1), lambda i: (0, 0)),
        out_shape=jax.ShapeDtypeStruct((G, 1), _f32),
        scratch_shapes=[
            pltpu.VMEM((G, HC), _f32),
            pltpu.VMEM((G, 8), _f32),
        ],
    )(acc, s, m, r8, batch3d, rt2d, hw8, hb8)


def _prep_layer(p):
    s = np.float32(1.0 / np.sqrt(C))
    we = p["We"]
    wep = jax.scipy.linalg.block_diag(
        *[we[:, h * C:(h + 1) * C].T for h in range(H)])
    wqp = p["Wq"] @ wep
    bqp = p["bq"] @ wep
    wcat = jnp.concatenate(
        [p["Wq"] * s, wqp * s, p["Wk"], p["Wv"], p["Ws"]], axis=1)
    bcat = jnp.concatenate(
        [p["bq"] * s, bqp * s, p["bk"], p["bv"], p["bs"]])[None, :]
    m = jax.scipy.linalg.block_diag(
        *[we[:, h * C:(h + 1) * C] for h in range(H)])
    return wcat, bcat, m


def kernel(x, edge_index, edge_attr, batch, r_target, params):
    npad = EPAD - E
    src = jnp.concatenate(
        [edge_index[0].astype(_i32), jnp.zeros((npad,), _i32)])
    dst = jnp.concatenate(
        [edge_index[1].astype(_i32), jnp.full((npad,), N, _i32)])
    ea_t = jnp.concatenate(
        [edge_attr.astype(_f32).T, jnp.zeros((DE, npad), _f32)], axis=1)
    zrs = jnp.zeros((NP // 16, ACC_W), _f32)
    r8 = jnp.concatenate(
        [jnp.repeat(jnp.eye(H, dtype=_f32), C, axis=1),
         jnp.zeros((4, HC), _f32)], axis=0)
    batch3d = batch.astype(_i32).reshape(_NBLK, 1, _BLK)
    rt2d = jnp.broadcast_to(r_target.astype(_i32)[:, None], (G, 8))
    hw8 = jnp.concatenate(
        [params["heads_W"][:, :, 0], jnp.zeros((4, HC), _f32)], axis=0)
    hb8 = jnp.zeros((8, 8), _f32).at[0:T, 0].set(params["heads_b"][:, 0])

    w1, b1, m1 = _prep_layer(params["conv1"])
    w2, b2, m2 = _prep_layer(params["conv2"])
    w3, b3, m3 = _prep_layer(params["conv3"])

    a, bb, s = _pre_call(x, w1, b1)
    acc = _edge_pass(a, bb, src, dst, ea_t, zrs)
    a, bb, s = _postpre_call(acc, s, m1, r8, w2, b2)
    acc = _edge_pass(a, bb, src, dst, ea_t, zrs)
    a, bb, s = _postpre_call(acc, s, m2, r8, w3, b3)
    acc = _edge_pass(a, bb, src, dst, ea_t, zrs)
    out = _final_call(acc, s, m3, r8, batch3d, rt2d, hw8, hb8)
    return out[:, 0]


# speedup vs baseline: 8.4707x; 1.0337x over previous
import functools

import jax
import jax.numpy as jnp
import numpy as np
from jax import lax
from jax.experimental import pallas as pl
from jax.experimental.pallas import tpu as pltpu
from jax.experimental.pallas import tpu_sc as plsc

N = 10000
E = 320000
D = 128
H = 4
C = 32
HC = H * C
G = 256
T = 4
DE = 4

ACC_W = 160
A_W = 144
B_W = 256
NP = 10240
NW = 32
BE = 128
SB = 32
EPAD = 323584
EW = EPAD // NW
NB = EW // BE
CH = SB // 16

_f32 = jnp.float32
_i32 = jnp.int32


def _edge_body(a_hbm, b_hbm, src_hbm, dst_hbm, ea_hbm, zrs_hbm, out_hbm,
               srcv_sb, dstv_sb, eav, arows, brows, orows, bounce, acc,
               sem0, sem1):
    cid = lax.axis_index("c")
    sid = lax.axis_index("s")
    rows_per_tile = NP // 16
    row0 = sid * rows_per_tile
    zv = jnp.zeros((16,), _f32)

    def zrow(r, _):
        def zcol(wd, _2):
            bounce[r, pl.ds(wd * 16, 16)] = zv
            return 0
        return lax.fori_loop(0, ACC_W // 16, zcol, 0)
    lax.fori_loop(0, SB, zrow, 0)

    def zcopy(i, _):
        pltpu.sync_copy(bounce, acc.at[pl.ds(row0 + i * SB, SB)])
        return 0
    lax.fori_loop(0, rows_per_tile // SB, zcopy, 0)
    plsc.subcore_barrier()

    w = sid * 2 + cid
    base_w = w * EW
    lane = lax.iota(_i32, 16)

    def block_body(b, _):
        base = base_w + b * BE
        pltpu.sync_copy(ea_hbm.at[:, pl.ds(base, BE)], eav)

        def sub_body(sb, _s0):
            sb0 = sb * SB
            i0 = pltpu.async_copy(src_hbm.at[pl.ds(base + sb0, SB)], srcv_sb,
                                  sem0)
            i1 = pltpu.async_copy(dst_hbm.at[pl.ds(base + sb0, SB)], dstv_sb,
                                  sem1)
            i0.wait()
            i1.wait()
            d0 = pltpu.async_copy(a_hbm.at[dstv_sb], arows, sem0)
            d1 = pltpu.async_copy(b_hbm.at[srcv_sb], brows, sem1)
            d0.wait()
            d1.wait()

            for j in range(CH):
                ridx = j * 16 + lane
                eac = [eav[d, pl.ds(sb0 + j * 16, 16)] for d in range(DE)]
                for h in range(H):
                    parts = [jnp.zeros((16,), _f32) for _ in range(4)]
                    for c2 in range(C):
                        col = jnp.full((16,), h * C + c2, _i32)
                        qc = plsc.load_gather(arows, [ridx, col])
                        kc = plsc.load_gather(brows, [ridx, col])
                        parts[c2 % 4] = parts[c2 % 4] + qc * kc
                    ah = (parts[0] + parts[1]) + (parts[2] + parts[3])
                    for d in range(DE):
                        qp = plsc.load_gather(
                            arows,
                            [ridx, jnp.full((16,), HC + h * DE + d, _i32)])
                        ah = ah + qp * eac[d]
                    ex_h = jnp.exp(ah)

                    for c2 in range(C):
                        vcol = jnp.full((16,), HC + h * C + c2, _i32)
                        ocol = jnp.full((16,), h * C + c2, _i32)
                        vc = plsc.load_gather(brows, [ridx, vcol])
                        plsc.store_scatter(orows, [ridx, ocol], vc * ex_h)
                    plsc.store_scatter(
                        orows, [ridx, jnp.full((16,), HC + h, _i32)], ex_h)
                    for d in range(DE):
                        plsc.store_scatter(
                            orows,
                            [ridx, jnp.full((16,), HC + 4 + h * DE + d, _i32)],
                            ex_h * eac[d])
            pltpu.sync_copy(orows, acc.at[dstv_sb], add=True)
            return 0

        lax.fori_loop(0, BE // SB, sub_body, 0)
        return 0

    lax.fori_loop(0, NB, block_body, 0)
    plsc.subcore_barrier()

    def ocopy(i, _):
        pltpu.sync_copy(acc.at[pl.ds(row0 + i * SB, SB)], bounce)
        pltpu.sync_copy(bounce, out_hbm.at[cid, pl.ds(row0 + i * SB, SB)])
        return 0
    lax.fori_loop(0, rows_per_tile // SB, ocopy, 0)


@functools.cache
def _edge_pass_fn():
    return pl.kernel(
        _edge_body,
        compiler_params=pltpu.CompilerParams(use_tc_tiling_on_sc=False,
                                             needs_layout_passes=False),
        out_type=jax.ShapeDtypeStruct((2, NP, ACC_W), _f32),
        mesh=plsc.VectorSubcoreMesh(core_axis_name="c", subcore_axis_name="s",
                                    num_cores=2, num_subcores=16),
        scratch_types=[
            pltpu.VMEM((SB,), _i32),
            pltpu.VMEM((SB,), _i32),
            pltpu.VMEM((DE, BE), _f32),
            pltpu.VMEM((SB, A_W), _f32),
            pltpu.VMEM((SB, B_W), _f32),
            pltpu.VMEM((SB, ACC_W), _f32),
            pltpu.VMEM((SB, ACC_W), _f32),
            pltpu.VMEM_SHARED((NP, ACC_W), _f32),
            pltpu.SemaphoreType.DMA,
            pltpu.SemaphoreType.DMA,
        ],
    )


def _edge_pass(a, bb, src, dst, ea_t, zrs):
    return _edge_pass_fn()(a, bb, src, dst, ea_t, zrs)


_BLK = 1000
_NBLK = N // _BLK


def _pre_body(x_ref, w_ref, b_ref, a_ref, bb_ref, s_ref):
    z = jnp.dot(x_ref[...], w_ref[...], preferred_element_type=_f32) + b_ref[...]
    a_ref[...] = z[:, 0:A_W]
    bb_ref[...] = z[:, A_W:A_W + B_W]
    s_ref[...] = z[:, A_W + B_W:]


def _pre_call(h, wcat, bcat):
    return pl.pallas_call(
        _pre_body,
        grid=(_NBLK,),
        in_specs=[
            pl.BlockSpec((_BLK, HC), lambda i: (i, 0)),
            pl.BlockSpec((HC, 528), lambda i: (0, 0)),
            pl.BlockSpec((1, 528), lambda i: (0, 0)),
        ],
        out_specs=[
            pl.BlockSpec((_BLK, A_W), lambda i: (i, 0)),
            pl.BlockSpec((_BLK, B_W), lambda i: (i, 0)),
            pl.BlockSpec((_BLK, HC), lambda i: (i, 0)),
        ],
        out_shape=[
            jax.ShapeDtypeStruct((NP, A_W), _f32),
            jax.ShapeDtypeStruct((NP, B_W), _f32),
            jax.ShapeDtypeStruct((N, HC), _f32),
        ],
    )(h, wcat, bcat)


def _node_out(acc_ref, s_ref, m_ref, r_ref):
    accs = acc_ref[0] + acc_ref[1]
    num = accs[:, 0:HC] + jnp.dot(accs[:, HC + 4:HC + 20], m_ref[...],
                                  preferred_element_type=_f32)
    den = jnp.dot(accs[:, HC:HC + 8], r_ref[...],
                  preferred_element_type=_f32) + 1e-16
    return num / den + s_ref[...]


def _postpre_body(acc_ref, s_ref, m_ref, r_ref, w_ref, b_ref,
                  a_ref, bb_ref, s2_ref):
    h = jnp.maximum(_node_out(acc_ref, s_ref, m_ref, r_ref), 0.0)
    z = jnp.dot(h, w_ref[...], preferred_element_type=_f32) + b_ref[...]
    a_ref[...] = z[:, 0:A_W]
    bb_ref[...] = z[:, A_W:A_W + B_W]
    s2_ref[...] = z[:, A_W + B_W:]


def _postpre_call(acc, s, m, r8, wcat, bcat):
    return pl.pallas_call(
        _postpre_body,
        grid=(_NBLK,),
        in_specs=[
            pl.BlockSpec((2, _BLK, ACC_W), lambda i: (0, i, 0)),
            pl.BlockSpec((_BLK, HC), lambda i: (i, 0)),
            pl.BlockSpec((16, HC), lambda i: (0, 0)),
            pl.BlockSpec((8, HC), lambda i: (0, 0)),
            pl.BlockSpec((HC, 528), lambda i: (0, 0)),
            pl.BlockSpec((1, 528), lambda i: (0, 0)),
        ],
        out_specs=[
            pl.BlockSpec((_BLK, A_W), lambda i: (i, 0)),
            pl.BlockSpec((_BLK, B_W), lambda i: (i, 0)),
            pl.BlockSpec((_BLK, HC), lambda i: (i, 0)),
        ],
        out_shape=[
            jax.ShapeDtypeStruct((NP, A_W), _f32),
            jax.ShapeDtypeStruct((NP, B_W), _f32),
            jax.ShapeDtypeStruct((N, HC), _f32),
        ],
    )(acc, s, m, r8, wcat, bcat)


def _final_body(acc_ref, s_ref, m_ref, r_ref, batch_ref, rt_ref,
                hw_ref, hb_ref, out_ref, pooled, cnt):
    i = pl.program_id(0)

    @pl.when(i == 0)
    def _init():
        pooled[...] = jnp.zeros_like(pooled)
        cnt[...] = jnp.zeros_like(cnt)

    h3 = _node_out(acc_ref, s_ref, m_ref, r_ref)
    b = batch_ref[0, 0, :]
    oh = (jnp.reshape(b, (_BLK, 1)) ==
          lax.broadcasted_iota(_i32, (_BLK, G), 1)).astype(_f32)
    dn = (((0,), (0,)), ((), ()))
    pooled[...] += lax.dot_general(oh, h3, dn, preferred_element_type=_f32)
    cnt[...] += lax.dot_general(oh, jnp.ones((_BLK, 8), _f32), dn,
                                preferred_element_type=_f32)

    @pl.when(i == _NBLK - 1)
    def _head():
        pm = pooled[...] / jnp.maximum(cnt[:, 0:1], 1.0)
        rt = rt_ref[:, 0:1]
        ohr = (rt == lax.broadcasted_iota(_i32, (G, 8), 1)).astype(_f32)
        whsel = jnp.dot(ohr, hw_ref[...], preferred_element_type=_f32)
        bhsel = jnp.dot(ohr, hb_ref[...], preferred_element_type=_f32)
        out_ref[...] = (jnp.sum(pm * whsel, axis=1, keepdims=True)
                        + bhsel[:, 0:1])


def _final_call(acc, s, m, r8, batch3d, rt2d, hw8, hb8):
    return pl.pallas_call(
        _final_body,
        grid=(_NBLK,),
        in_specs=[
            pl.BlockSpec((2, _BLK, ACC_W), lambda i: (0, i, 0)),
            pl.BlockSpec((_BLK, HC), lambda i: (i, 0)),
            pl.BlockSpec((16, HC), lambda i: (0, 0)),
            pl.BlockSpec((8, HC), lambda i: (0, 0)),
            pl.BlockSpec((1, 1, _BLK), lambda i: (i, 0, 0)),
            pl.BlockSpec((G, 8), lambda i: (0, 0)),
            pl.BlockSpec((8, HC), lambda i: (0, 0)),
            pl.BlockSpec((8, 8), lambda i: (0, 0)),
        ],
        out_specs=pl.BlockSpec((G, 1), lambda i: (0, 0)),
        out_shape=jax.ShapeDtypeStruct((G, 1), _f32),
        scratch_shapes=[
            pltpu.VMEM((G, HC), _f32),
            pltpu.VMEM((G, 8), _f32),
        ],
    )(acc, s, m, r8, batch3d, rt2d, hw8, hb8)


def _prep_layer(p):
    s = np.float32(1.0 / np.sqrt(C))
    we = p["We"]
    wep = jax.scipy.linalg.block_diag(
        *[we[:, h * C:(h + 1) * C].T for h in range(H)])
    wqp = p["Wq"] @ wep
    bqp = p["bq"] @ wep
    wcat = jnp.concatenate(
        [p["Wq"] * s, wqp * s, p["Wk"], p["Wv"], p["Ws"]], axis=1)
    bcat = jnp.concatenate(
        [p["bq"] * s, bqp * s, p["bk"], p["bv"], p["bs"]])[None, :]
    m = jax.scipy.linalg.block_diag(
        *[we[:, h * C:(h + 1) * C] for h in range(H)])
    return wcat, bcat, m


def kernel(x, edge_index, edge_attr, batch, r_target, params):
    npad = EPAD - E
    src = jnp.concatenate(
        [edge_index[0].astype(_i32), jnp.zeros((npad,), _i32)])
    dst = jnp.concatenate(
        [edge_index[1].astype(_i32), jnp.full((npad,), N, _i32)])
    ea_t = jnp.concatenate(
        [edge_attr.astype(_f32).T, jnp.zeros((DE, npad), _f32)], axis=1)
    zrs = jnp.zeros((NP // 16, ACC_W), _f32)
    r8 = jnp.concatenate(
        [jnp.repeat(jnp.eye(H, dtype=_f32), C, axis=1),
         jnp.zeros((4, HC), _f32)], axis=0)
    batch3d = batch.astype(_i32).reshape(_NBLK, 1, _BLK)
    rt2d = jnp.broadcast_to(r_target.astype(_i32)[:, None], (G, 8))
    hw8 = jnp.concatenate(
        [params["heads_W"][:, :, 0], jnp.zeros((4, HC), _f32)], axis=0)
    hb8 = jnp.zeros((8, 8), _f32).at[0:T, 0].set(params["heads_b"][:, 0])

    w1, b1, m1 = _prep_layer(params["conv1"])
    w2, b2, m2 = _prep_layer(params["conv2"])
    w3, b3, m3 = _prep_layer(params["conv3"])

    a, bb, s = _pre_call(x, w1, b1)
    acc = _edge_pass(a, bb, src, dst, ea_t, zrs)
    a, bb, s = _postpre_call(acc, s, m1, r8, w2, b2)
    acc = _edge_pass(a, bb, src, dst, ea_t, zrs)
    a, bb, s = _postpre_call(acc, s, m2, r8, w3, b3)
    acc = _edge_pass(a, bb, src, dst, ea_t, zrs)
    out = _final_call(acc, s, m3, r8, batch3d, rt2d, hw8, hb8)
    return out[:, 0]
